# baseline (speedup 1.0000x reference)
import functools

import jax
import jax.numpy as jnp
from jax import lax
from jax.experimental import pallas as pl
from jax.experimental.pallas import tpu as pltpu
from jax.experimental.pallas import tpu_sc as plsc


def kernel(token_id, table):
    B = token_id.shape[0]
    V, D = table.shape
    table128 = table.reshape(V // 4, 4 * D)
    info = plsc.get_sparse_core_info()
    NC, NS, L = info.num_cores, info.num_subcores, info.num_lanes
    NW = NC * NS
    assert B % (8 * NW) == 0
    b_per_w = B // NW
    CHUNK = 128
    mesh = plsc.VectorSubcoreMesh(core_axis_name="c", subcore_axis_name="s")

    @functools.partial(
        pl.kernel,
        mesh=mesh,
        out_type=jax.ShapeDtypeStruct((B, D), jnp.float32),
        scratch_types=[
            pltpu.VMEM((b_per_w,), jnp.int32),
            pltpu.VMEM((b_per_w,), jnp.int32),
            pltpu.VMEM((2, CHUNK, 4 * D), jnp.float32),
            pltpu.VMEM((b_per_w, D), jnp.float32),
            pltpu.SemaphoreType.DMA,
        ],
        compiler_params=pltpu.CompilerParams(needs_layout_passes=False),
    )
    def gather_kernel(idx_hbm, t128_hbm, out_hbm, idx_v, row_v, rows_v, out_v, sem):
        wid = lax.axis_index("s") * NC + lax.axis_index("c")
        base = wid * b_per_w
        n_chunks = b_per_w // CHUNK

        pltpu.sync_copy(idx_hbm.at[pl.ds(base, b_per_w)], idx_v)

        def rowidx(g, carry):
            v = idx_v[pl.ds(g * L, L)]
            row_v[pl.ds(g * L, L)] = v >> 2
            return carry

        lax.fori_loop(0, b_per_w // L, rowidx, None)

        def start_chunk(k):
            pltpu.make_async_copy(
                t128_hbm.at[row_v.at[pl.ds(k * CHUNK, CHUNK)]],
                rows_v.at[k % 2],
                sem,
            ).start()

        def wait_chunk():
            pltpu.make_async_copy(
                t128_hbm.at[row_v.at[pl.ds(0, CHUNK)]],
                rows_v.at[0],
                sem,
            ).wait()

        lanes = lax.iota(jnp.int32, L)

        def extract_chunk(k):
            buf = rows_v.at[k % 2]

            def extract(g, carry):
                toks = idx_v[pl.ds(k * CHUNK + g * L, L)]
                col_base = (toks & 3) * D
                i_loc = g * L + lanes
                i_dst = k * CHUNK + g * L + lanes

                def col(j, carry2):
                    vals = plsc.load_gather(buf, [i_loc, col_base + j])
                    plsc.store_scatter(
                        out_v, [i_dst, jnp.zeros((L,), jnp.int32) + j], vals
                    )
                    return carry2

                lax.fori_loop(0, D, col, None)
                return carry

            lax.fori_loop(0, CHUNK // L, extract, None)

        start_chunk(0)
        for k in range(n_chunks):
            wait_chunk()
            if k + 1 < n_chunks:
                start_chunk(k + 1)
            extract_chunk(k)

        pltpu.sync_copy(out_v, out_hbm.at[pl.ds(base, b_per_w)])

    return gather_kernel(token_id.astype(jnp.int32), table128)


# speedup vs baseline: 1.7012x; 1.7012x over previous
import functools

import jax
import jax.numpy as jnp
from jax import lax
from jax.experimental import pallas as pl
from jax.experimental.pallas import tpu as pltpu
from jax.experimental.pallas import tpu_sc as plsc


def kernel(token_id, table):
    B = token_id.shape[0]
    V, D = table.shape
    info = plsc.get_sparse_core_info()
    NC, NS, L = info.num_cores, info.num_subcores, info.num_lanes
    NW = NC * NS
    assert B % (8 * NW) == 0
    b_per_w = B // NW
    LAG = 7
    mesh = plsc.VectorSubcoreMesh(core_axis_name="c", subcore_axis_name="s")

    @functools.partial(
        pl.kernel,
        mesh=mesh,
        out_type=jax.ShapeDtypeStruct((B, D), jnp.float32),
        scratch_types=[
            pltpu.VMEM((b_per_w,), jnp.int32),
            pltpu.VMEM((b_per_w, D), jnp.float32),
            pltpu.SemaphoreType.DMA,
        ],
    )
    def gather_kernel(idx_hbm, table_hbm, out_hbm, idx_v, rows_v, sem):
        wid = lax.axis_index("s") * NC + lax.axis_index("c")
        base = wid * b_per_w
        pltpu.sync_copy(idx_hbm.at[pl.ds(base, b_per_w)], idx_v)

        def drain_one(i, carry):
            pltpu.make_async_copy(
                table_hbm.at[pl.ds(0, 1)],
                rows_v.at[pl.ds(0, 1)],
                sem,
            ).wait()
            return carry

        def group(g, carry):
            toks = idx_v[pl.ds(g * L, L)]
            for j in range(L):
                pltpu.make_async_copy(
                    table_hbm.at[pl.ds(toks[j], 1)],
                    rows_v.at[pl.ds(g * L + j, 1)],
                    sem,
                ).start()
            lax.cond(
                g >= LAG,
                lambda: lax.fori_loop(0, L, drain_one, None),
                lambda: None,
            )
            return carry

        lax.fori_loop(0, b_per_w // L, group, None)
        lax.fori_loop(0, LAG * L, drain_one, None)

        pltpu.sync_copy(rows_v, out_hbm.at[pl.ds(base, b_per_w)])

    return gather_kernel(token_id.astype(jnp.int32), table)
